# baseline (speedup 1.0000x reference)
import dataclasses
import functools

import jax
import jax.numpy as jnp
from jax import lax
from jax.experimental import pallas as pl
from jax.experimental.pallas import tpu as pltpu
from jax.experimental.pallas import tpu_sc as plsc

N_NODES = 10000
N_USER = 5000
N_EDGES = 320000
D = 128
B = 4096

NC = 2
NS = 16
LANES = 16
CHUNK = 128
NBUF = 2
NIB = 4
SUPER = 4
CHUNKS_PER_WORKER = 80
N_CHUNKS = NC * NS * CHUNKS_PER_WORKER
ROWS_PER_SUBCORE = 624


def _mesh():
    return plsc.VectorSubcoreMesh(core_axis_name="c", subcore_axis_name="s")


def _sc_compiler_params():
    cp = pltpu.CompilerParams()
    fields = pltpu.CompilerParams.__dataclass_fields__
    if "needs_layout_passes" in fields:
        cp = dataclasses.replace(cp, needs_layout_passes=False)
    if "use_tc_tiling_on_sc" in fields:
        cp = dataclasses.replace(cp, use_tc_tiling_on_sc=False)
    return cp


def _sc_spmm(ei, ev, table):
    @functools.partial(
        pl.kernel,
        out_type=jax.ShapeDtypeStruct((NC, N_NODES, D), jnp.float32),
        mesh=_mesh(),
        scratch_types=[
            [pltpu.VMEM((2, CHUNK), jnp.int32) for _ in range(NIB)],
            [pltpu.VMEM((CHUNK,), jnp.float32) for _ in range(2)],
            [pltpu.VMEM((CHUNK, D // 2), jnp.int32) for _ in range(NBUF)],
            [pltpu.VMEM((CHUNK, D), jnp.float32) for _ in range(2)],
            [pltpu.SemaphoreType.DMA for _ in range(NIB)],
            [pltpu.SemaphoreType.DMA for _ in range(2)],
            [pltpu.SemaphoreType.DMA for _ in range(NBUF)],
            [pltpu.SemaphoreType.DMA for _ in range(2)],
            pltpu.VMEM_SHARED((N_NODES, D), jnp.float32),
        ],
        compiler_params=_sc_compiler_params(),
    )
    def spmm(ei_h, ev_h, tab_h, out_h, ibs, evs, gbufs, sbufs,
             ibsems, evsems, gsems, ssems, acc):
        c = lax.axis_index("c")
        s = lax.axis_index("s")
        wbase = (c * NS + s) * CHUNKS_PER_WORKER

        def ibload(i, slot):
            pltpu.async_copy(ei_h.at[:, pl.ds((wbase + i) * CHUNK, CHUNK)],
                             ibs[slot], ibsems[slot])

        def ibwait(i, slot):
            pltpu.make_async_copy(ei_h.at[:, pl.ds((wbase + i) * CHUNK, CHUNK)],
                                  ibs[slot], ibsems[slot]).wait()

        def evload(i, e):
            pltpu.async_copy(ev_h.at[pl.ds((wbase + i) * CHUNK, CHUNK)],
                             evs[e], evsems[e])

        def evwait(i, e):
            pltpu.make_async_copy(ev_h.at[pl.ds((wbase + i) * CHUNK, CHUNK)],
                                  evs[e], evsems[e]).wait()

        def gather(g, slot):
            pltpu.async_copy(tab_h.at[ibs[slot].at[1]], gbufs[g], gsems[g])

        def gwait(g, slot):
            pltpu.make_async_copy(tab_h.at[ibs[slot].at[1]], gbufs[g],
                                  gsems[g]).wait()

        def scatter(sb, slot):
            pltpu.async_copy(sbufs[sb], acc.at[ibs[slot].at[0]], ssems[sb],
                             add=True)

        def swait(sb, slot):
            pltpu.make_async_copy(sbufs[sb], acc.at[ibs[slot].at[0]],
                                  ssems[sb]).wait()

        ibload(0, 0)
        ibload(1, 1)
        evload(0, 0)
        evload(1, 1)

        zero16 = jnp.zeros((LANES,), jnp.float32)

        @pl.loop(0, CHUNK)
        def _(k):
            for d in range(D // LANES):
                sbufs[0][k, pl.ds(d * LANES, LANES)] = zero16

        base_row = s * ROWS_PER_SUBCORE
        for j in range(4):
            pltpu.sync_copy(sbufs[0].at[pl.ds(0, CHUNK)],
                            acc.at[pl.ds(base_row + j * CHUNK, CHUNK)])
        pltpu.sync_copy(sbufs[0].at[pl.ds(0, 112)],
                        acc.at[pl.ds(base_row + 4 * CHUNK, 112)])

        @pl.when(s == NS - 1)
        def _():
            pltpu.sync_copy(sbufs[0].at[pl.ds(0, 16)],
                            acc.at[pl.ds(N_NODES - 16, 16)])

        plsc.subcore_barrier()

        himask = jnp.full((LANES,), -65536, jnp.int32)

        def scale(g, sb, e):
            @pl.loop(0, CHUNK)
            def _(k):
                evk = plsc.load_gather(
                    evs[e], [jnp.full((LANES,), k, jnp.int32)])
                for t in range(D // (2 * LANES)):
                    v = gbufs[g][k, pl.ds(t * LANES, LANES)]
                    lo = plsc.bitcast(v << 16, jnp.float32)
                    hi = plsc.bitcast(v & himask, jnp.float32)
                    sbufs[sb][k, pl.ds(t * LANES, LANES)] = lo * evk
                    sbufs[sb][k, pl.ds(D // 2 + t * LANES, LANES)] = hi * evk

        ibwait(0, 0)
        gather(0, 0)

        @pl.loop(0, CHUNKS_PER_WORKER, step=SUPER)
        def _(i):
            for j in range(SUPER):
                ch = i + j
                g, sb, e, slot = j % 2, j % 2, j % 2, j % NIB
                sbp, slotp = j % 2, (j - 2) % NIB
                gn, slotn = (j + 1) % 2, (j + 1) % NIB
                slotl = (j + 2) % NIB
                gwait(g, slot)

                @pl.when(ch >= 2)
                def _():
                    swait(sbp, slotp)

                @pl.when(ch + 1 < CHUNKS_PER_WORKER)
                def _():
                    ibwait(ch + 1, slotn)
                    gather(gn, slotn)

                @pl.when(ch + 2 < CHUNKS_PER_WORKER)
                def _():
                    ibload(ch + 2, slotl)

                evwait(ch, e)
                scale(g, sb, e)
                scatter(sb, slot)

                @pl.when(ch + 2 < CHUNKS_PER_WORKER)
                def _():
                    evload(ch + 2, e)

        swait((CHUNKS_PER_WORKER - 2) % 2, (CHUNKS_PER_WORKER - 2) % NIB)
        swait((CHUNKS_PER_WORKER - 1) % 2, (CHUNKS_PER_WORKER - 1) % NIB)

        plsc.subcore_barrier()
        pltpu.sync_copy(acc.at[pl.ds(base_row, ROWS_PER_SUBCORE)],
                        out_h.at[c, pl.ds(base_row, ROWS_PER_SUBCORE)])

        @pl.when(s == NS - 1)
        def _():
            pltpu.sync_copy(acc.at[pl.ds(N_NODES - 16, 16)],
                            out_h.at[c, pl.ds(N_NODES - 16, 16)])

    return spmm(ei, ev, table)


def _pack_halves(h):
    lo = jax.lax.bitcast_convert_type(
        h[:, :D // 2].astype(jnp.bfloat16), jnp.uint16).astype(jnp.uint32)
    hi = jax.lax.bitcast_convert_type(
        h[:, D // 2:].astype(jnp.bfloat16), jnp.uint16).astype(jnp.uint32)
    return jax.lax.bitcast_convert_type(lo | (hi << 16), jnp.int32)


def _tc_add(p):
    def body(p_ref, o_ref):
        o_ref[...] = _pack_halves(p_ref[0] + p_ref[1])

    return pl.pallas_call(
        body,
        out_shape=jax.ShapeDtypeStruct((N_NODES, D // 2), jnp.int32),
    )(p)


def _softplus(t):
    return jnp.maximum(t, 0.0) + jnp.log(1.0 + jnp.exp(-jnp.abs(t)))


ROW_BLK = 1000


def _tc_encode(q, noise, W_mean, b_mean, W_std, b_std):
    nblk = N_NODES // ROW_BLK

    def body(q_ref, n_ref, wm_ref, bm_ref, ws_ref, bs_ref, x_ref, kl_ref, acc_ref):
        i = pl.program_id(0)
        h2 = q_ref[0].astype(jnp.float32) + q_ref[1].astype(jnp.float32)
        m = jnp.dot(h2, wm_ref[...], preferred_element_type=jnp.float32) + bm_ref[...]
        t = jnp.dot(h2, ws_ref[...], preferred_element_type=jnp.float32) + bs_ref[...]
        sstd = _softplus(t)
        x_ref[...] = m + n_ref[...] * sstd
        blk_kl = jnp.sum(1.0 + 2.0 * jnp.log(sstd + 1e-8) - m * m - sstd * sstd)

        @pl.when(i == 0)
        def _():
            acc_ref[0] = 0.0

        acc_ref[0] += blk_kl

        @pl.when(i == nblk - 1)
        def _():
            kl_ref[...] = jnp.full((1, 1), -0.5 / N_NODES, jnp.float32) * acc_ref[0]

    return pl.pallas_call(
        body,
        grid=(nblk,),
        in_specs=[
            pl.BlockSpec((2, ROW_BLK, D), lambda i: (0, i, 0)),
            pl.BlockSpec((ROW_BLK, D), lambda i: (i, 0)),
            pl.BlockSpec((D, D), lambda i: (0, 0)),
            pl.BlockSpec((1, D), lambda i: (0, 0)),
            pl.BlockSpec((D, D), lambda i: (0, 0)),
            pl.BlockSpec((1, D), lambda i: (0, 0)),
        ],
        out_specs=[
            pl.BlockSpec((ROW_BLK, D), lambda i: (i, 0)),
            pl.BlockSpec((1, 1), lambda i: (0, 0)),
        ],
        out_shape=[
            jax.ShapeDtypeStruct((N_NODES, D), jnp.float32),
            jax.ShapeDtypeStruct((1, 1), jnp.float32),
        ],
        scratch_shapes=[pltpu.SMEM((1,), jnp.float32)],
    )(q, noise, W_mean, b_mean, W_std, b_std)


BPW = B // (NC * NS)


def _sc_decode(x, users, items, neg_items):
    @functools.partial(
        pl.kernel,
        out_type=(jax.ShapeDtypeStruct((B, D), jnp.float32),
                  jax.ShapeDtypeStruct((B, D), jnp.float32)),
        mesh=_mesh(),
        scratch_types=[
            pltpu.VMEM((BPW,), jnp.int32),
            pltpu.VMEM((BPW,), jnp.int32),
            pltpu.VMEM((BPW,), jnp.int32),
            pltpu.VMEM((BPW, D), jnp.float32),
            pltpu.VMEM((BPW, D), jnp.float32),
            pltpu.VMEM((BPW, D), jnp.float32),
            [pltpu.SemaphoreType.DMA for _ in range(3)],
        ],
        compiler_params=_sc_compiler_params(),
    )
    def dec(x_h, u_h, it_h, ng_h, zp_h, zn_h, uv, iv, nv, xu, xi, xn, sems):
        c = lax.axis_index("c")
        s = lax.axis_index("s")
        base = (s * NC + c) * BPW
        pltpu.sync_copy(u_h.at[pl.ds(base, BPW)], uv)
        pltpu.sync_copy(it_h.at[pl.ds(base, BPW)], iv)
        pltpu.sync_copy(ng_h.at[pl.ds(base, BPW)], nv)
        off = jnp.full((LANES,), N_USER, jnp.int32)

        @pl.loop(0, BPW // LANES)
        def _(j):
            sl = pl.ds(j * LANES, LANES)
            iv[sl] = iv[sl] + off
            nv[sl] = nv[sl] + off

        du = pltpu.async_copy(x_h.at[uv], xu, sems[0])
        di = pltpu.async_copy(x_h.at[iv], xi, sems[1])
        dn = pltpu.async_copy(x_h.at[nv], xn, sems[2])
        du.wait()
        di.wait()
        dn.wait()

        @pl.loop(0, BPW)
        def _(r):
            for d in range(D // LANES):
                sl = pl.ds(d * LANES, LANES)
                u = xu[r, sl]
                xi[r, sl] = u * xi[r, sl]
                xn[r, sl] = u * xn[r, sl]

        pltpu.sync_copy(xi, zp_h.at[pl.ds(base, BPW)])
        pltpu.sync_copy(xn, zn_h.at[pl.ds(base, BPW)])

    return dec(x, users, items, neg_items)


def _tc_loss(zp, zn, Wd1, bd1, wd2, bd2, kl):
    def body(zp_ref, zn_ref, w1_ref, b1_ref, w2_ref, b2_ref, kl_ref, o_ref):
        w1 = w1_ref[...]
        b1 = b1_ref[...]
        w2 = w2_ref[...]
        hp = jnp.maximum(jnp.dot(zp_ref[...], w1, preferred_element_type=jnp.float32) + b1, 0.0)
        hn = jnp.maximum(jnp.dot(zn_ref[...], w1, preferred_element_type=jnp.float32) + b1, 0.0)
        lp = jnp.sum(hp * w2, axis=1, keepdims=True) + b2_ref[0, 0]
        ln = jnp.sum(hn * w2, axis=1, keepdims=True) + b2_ref[0, 0]
        lr = jnp.mean(_softplus(-lp)) + jnp.mean(_softplus(ln))
        o_ref[...] = jnp.full((1, 1), 1.0, jnp.float32) * (lr + 0.1 * kl_ref[0, 0])

    return pl.pallas_call(
        body,
        out_shape=jax.ShapeDtypeStruct((1, 1), jnp.float32),
    )(zp, zn, Wd1, bd1, wd2, bd2, kl)


def kernel(edge_values, emb, W_mean, b_mean, W_std, b_std, Wd1, bd1, Wd2, bd2,
           noise, edge_index, users, items, neg_items):
    pad = N_CHUNKS * CHUNK - N_EDGES
    ei = jnp.concatenate([edge_index, jnp.zeros((2, pad), jnp.int32)], axis=1)
    ev = jnp.concatenate([edge_values, jnp.zeros((pad,), jnp.float32)])
    p = _sc_spmm(ei, ev, _pack_halves(emb))
    h1 = _tc_add(p)
    q = _sc_spmm(ei, ev, h1)
    x, kl = _tc_encode(q, noise, W_mean, b_mean.reshape(1, D),
                       W_std, b_std.reshape(1, D))
    zp, zn = _sc_decode(x, users, items, neg_items)
    loss = _tc_loss(zp, zn, Wd1, bd1.reshape(1, D), Wd2.reshape(1, D),
                    bd2.reshape(1, 1), kl)
    return loss[0, 0]


# speedup vs baseline: 1.2561x; 1.2561x over previous
import dataclasses
import functools

import jax
import jax.numpy as jnp
from jax import lax
from jax.experimental import pallas as pl
from jax.experimental.pallas import tpu as pltpu
from jax.experimental.pallas import tpu_sc as plsc

N_NODES = 10000
N_USER = 5000
N_EDGES = 320000
D = 128
B = 4096

NC = 2
NS = 16
LANES = 16
CHUNK = 128
NBUF = 2
NIB = 4
SUPER = 4
CHUNKS_PER_WORKER = 80
N_CHUNKS = NC * NS * CHUNKS_PER_WORKER
ROWS_PER_SUBCORE = 624


def _mesh():
    return plsc.VectorSubcoreMesh(core_axis_name="c", subcore_axis_name="s")


def _sc_compiler_params():
    cp = pltpu.CompilerParams()
    fields = pltpu.CompilerParams.__dataclass_fields__
    if "needs_layout_passes" in fields:
        cp = dataclasses.replace(cp, needs_layout_passes=False)
    if "use_tc_tiling_on_sc" in fields:
        cp = dataclasses.replace(cp, use_tc_tiling_on_sc=False)
    return cp


def _sc_spmm(packed, table):
    @functools.partial(
        pl.kernel,
        out_type=jax.ShapeDtypeStruct((NC, N_NODES, D), jnp.float32),
        mesh=_mesh(),
        scratch_types=[
            [pltpu.VMEM((2, CHUNK), jnp.int32) for _ in range(NIB)],
            [pltpu.VMEM((CHUNK,), jnp.int32) for _ in range(2)],
            [pltpu.VMEM((CHUNK, D // 2), jnp.int32) for _ in range(NBUF)],
            [pltpu.VMEM((CHUNK, D), jnp.float32) for _ in range(2)],
            [pltpu.SemaphoreType.DMA for _ in range(NIB)],
            [pltpu.SemaphoreType.DMA for _ in range(2)],
            [pltpu.SemaphoreType.DMA for _ in range(NBUF)],
            [pltpu.SemaphoreType.DMA for _ in range(2)],
            pltpu.VMEM_SHARED((N_NODES, D), jnp.float32),
        ],
        compiler_params=_sc_compiler_params(),
    )
    def spmm(pk_h, tab_h, out_h, ibs, evs, gbufs, sbufs,
             ibsems, evsems, gsems, ssems, acc):
        c = lax.axis_index("c")
        s = lax.axis_index("s")
        wbase = (c * NS + s) * CHUNKS_PER_WORKER

        def ibload(i, slot):
            pltpu.async_copy(pk_h.at[wbase + i, pl.ds(0, 2)], ibs[slot],
                             ibsems[slot])

        def ibwait(i, slot):
            pltpu.make_async_copy(pk_h.at[wbase + i, pl.ds(0, 2)], ibs[slot],
                                  ibsems[slot]).wait()

        def evload(i, e):
            pltpu.async_copy(pk_h.at[wbase + i, 2], evs[e], evsems[e])

        def evwait(i, e):
            pltpu.make_async_copy(pk_h.at[wbase + i, 2], evs[e],
                                  evsems[e]).wait()

        def gather(g, slot):
            pltpu.async_copy(tab_h.at[ibs[slot].at[1]], gbufs[g], gsems[g])

        def gwait(g, slot):
            pltpu.make_async_copy(tab_h.at[ibs[slot].at[1]], gbufs[g],
                                  gsems[g]).wait()

        def scatter(sb, slot):
            pltpu.async_copy(sbufs[sb], acc.at[ibs[slot].at[0]], ssems[sb],
                             add=True)

        def swait(sb, slot):
            pltpu.make_async_copy(sbufs[sb], acc.at[ibs[slot].at[0]],
                                  ssems[sb]).wait()

        ibload(0, 0)
        ibload(1, 1)
        evload(0, 0)
        evload(1, 1)

        zero16 = jnp.zeros((LANES,), jnp.float32)

        @pl.loop(0, CHUNK)
        def _(k):
            for d in range(D // LANES):
                sbufs[0][k, pl.ds(d * LANES, LANES)] = zero16

        base_row = s * ROWS_PER_SUBCORE
        for j in range(4):
            pltpu.sync_copy(sbufs[0].at[pl.ds(0, CHUNK)],
                            acc.at[pl.ds(base_row + j * CHUNK, CHUNK)])
        pltpu.sync_copy(sbufs[0].at[pl.ds(0, 112)],
                        acc.at[pl.ds(base_row + 4 * CHUNK, 112)])

        @pl.when(s == NS - 1)
        def _():
            pltpu.sync_copy(sbufs[0].at[pl.ds(0, 16)],
                            acc.at[pl.ds(N_NODES - 16, 16)])

        plsc.subcore_barrier()

        himask = jnp.full((LANES,), -65536, jnp.int32)

        def scale(g, sb, e):
            @pl.loop(0, CHUNK)
            def _(k):
                evk = plsc.bitcast(
                    plsc.load_gather(evs[e], [jnp.full((LANES,), k, jnp.int32)]),
                    jnp.float32)
                for t in range(D // (2 * LANES)):
                    v = gbufs[g][k, pl.ds(t * LANES, LANES)]
                    lo = plsc.bitcast(v << 16, jnp.float32)
                    hi = plsc.bitcast(v & himask, jnp.float32)
                    sbufs[sb][k, pl.ds(t * LANES, LANES)] = lo * evk
                    sbufs[sb][k, pl.ds(D // 2 + t * LANES, LANES)] = hi * evk

        ibwait(0, 0)
        gather(0, 0)

        @pl.loop(0, CHUNKS_PER_WORKER, step=SUPER)
        def _(i):
            for j in range(SUPER):
                ch = i + j
                g, sb, e, slot = j % 2, j % 2, j % 2, j % NIB
                sbp, slotp = j % 2, (j - 2) % NIB
                gn, slotn = (j + 1) % 2, (j + 1) % NIB
                slotl = (j + 2) % NIB
                gwait(g, slot)

                @pl.when(ch >= 2)
                def _():
                    swait(sbp, slotp)

                @pl.when(ch + 1 < CHUNKS_PER_WORKER)
                def _():
                    ibwait(ch + 1, slotn)
                    gather(gn, slotn)

                @pl.when(ch + 2 < CHUNKS_PER_WORKER)
                def _():
                    ibload(ch + 2, slotl)

                evwait(ch, e)
                scale(g, sb, e)
                scatter(sb, slot)

                @pl.when(ch + 2 < CHUNKS_PER_WORKER)
                def _():
                    evload(ch + 2, e)

        swait((CHUNKS_PER_WORKER - 2) % 2, (CHUNKS_PER_WORKER - 2) % NIB)
        swait((CHUNKS_PER_WORKER - 1) % 2, (CHUNKS_PER_WORKER - 1) % NIB)

        plsc.subcore_barrier()
        pltpu.sync_copy(acc.at[pl.ds(base_row, ROWS_PER_SUBCORE)],
                        out_h.at[c, pl.ds(base_row, ROWS_PER_SUBCORE)])

        @pl.when(s == NS - 1)
        def _():
            pltpu.sync_copy(acc.at[pl.ds(N_NODES - 16, 16)],
                            out_h.at[c, pl.ds(N_NODES - 16, 16)])

    return spmm(packed, table)


def _pack_halves(h):
    lo = jax.lax.bitcast_convert_type(
        h[:, :D // 2].astype(jnp.bfloat16), jnp.uint16).astype(jnp.uint32)
    hi = jax.lax.bitcast_convert_type(
        h[:, D // 2:].astype(jnp.bfloat16), jnp.uint16).astype(jnp.uint32)
    return jax.lax.bitcast_convert_type(lo | (hi << 16), jnp.int32)


def _tc_add(p):
    def body(p_ref, o_ref):
        o_ref[...] = _pack_halves(p_ref[0] + p_ref[1])

    return pl.pallas_call(
        body,
        out_shape=jax.ShapeDtypeStruct((N_NODES, D // 2), jnp.int32),
    )(p)


def _softplus(t):
    return jnp.maximum(t, 0.0) + jnp.log(1.0 + jnp.exp(-jnp.abs(t)))


ROW_BLK = 1000


def _tc_encode(q, noise, W_mean, b_mean, W_std, b_std):
    nblk = N_NODES // ROW_BLK

    def body(q_ref, n_ref, wm_ref, bm_ref, ws_ref, bs_ref, x_ref, kl_ref, acc_ref):
        i = pl.program_id(0)
        h2 = q_ref[0].astype(jnp.float32) + q_ref[1].astype(jnp.float32)
        m = jnp.dot(h2, wm_ref[...], preferred_element_type=jnp.float32) + bm_ref[...]
        t = jnp.dot(h2, ws_ref[...], preferred_element_type=jnp.float32) + bs_ref[...]
        sstd = _softplus(t)
        x_ref[...] = m + n_ref[...] * sstd
        blk_kl = jnp.sum(1.0 + 2.0 * jnp.log(sstd + 1e-8) - m * m - sstd * sstd)

        @pl.when(i == 0)
        def _():
            acc_ref[0] = 0.0

        acc_ref[0] += blk_kl

        @pl.when(i == nblk - 1)
        def _():
            kl_ref[...] = jnp.full((1, 1), -0.5 / N_NODES, jnp.float32) * acc_ref[0]

    return pl.pallas_call(
        body,
        grid=(nblk,),
        in_specs=[
            pl.BlockSpec((2, ROW_BLK, D), lambda i: (0, i, 0)),
            pl.BlockSpec((ROW_BLK, D), lambda i: (i, 0)),
            pl.BlockSpec((D, D), lambda i: (0, 0)),
            pl.BlockSpec((1, D), lambda i: (0, 0)),
            pl.BlockSpec((D, D), lambda i: (0, 0)),
            pl.BlockSpec((1, D), lambda i: (0, 0)),
        ],
        out_specs=[
            pl.BlockSpec((ROW_BLK, D), lambda i: (i, 0)),
            pl.BlockSpec((1, 1), lambda i: (0, 0)),
        ],
        out_shape=[
            jax.ShapeDtypeStruct((N_NODES, D), jnp.float32),
            jax.ShapeDtypeStruct((1, 1), jnp.float32),
        ],
        scratch_shapes=[pltpu.SMEM((1,), jnp.float32)],
    )(q, noise, W_mean, b_mean, W_std, b_std)


BPW = B // (NC * NS)


def _sc_decode(x, users, items, neg_items):
    @functools.partial(
        pl.kernel,
        out_type=(jax.ShapeDtypeStruct((B, D), jnp.float32),
                  jax.ShapeDtypeStruct((B, D), jnp.float32)),
        mesh=_mesh(),
        scratch_types=[
            pltpu.VMEM((BPW,), jnp.int32),
            pltpu.VMEM((BPW,), jnp.int32),
            pltpu.VMEM((BPW,), jnp.int32),
            pltpu.VMEM((BPW, D), jnp.float32),
            pltpu.VMEM((BPW, D), jnp.float32),
            pltpu.VMEM((BPW, D), jnp.float32),
            [pltpu.SemaphoreType.DMA for _ in range(3)],
        ],
        compiler_params=_sc_compiler_params(),
    )
    def dec(x_h, u_h, it_h, ng_h, zp_h, zn_h, uv, iv, nv, xu, xi, xn, sems):
        c = lax.axis_index("c")
        s = lax.axis_index("s")
        base = (s * NC + c) * BPW
        pltpu.sync_copy(u_h.at[pl.ds(base, BPW)], uv)
        pltpu.sync_copy(it_h.at[pl.ds(base, BPW)], iv)
        pltpu.sync_copy(ng_h.at[pl.ds(base, BPW)], nv)
        off = jnp.full((LANES,), N_USER, jnp.int32)

        @pl.loop(0, BPW // LANES)
        def _(j):
            sl = pl.ds(j * LANES, LANES)
            iv[sl] = iv[sl] + off
            nv[sl] = nv[sl] + off

        du = pltpu.async_copy(x_h.at[uv], xu, sems[0])
        di = pltpu.async_copy(x_h.at[iv], xi, sems[1])
        dn = pltpu.async_copy(x_h.at[nv], xn, sems[2])
        du.wait()
        di.wait()
        dn.wait()

        @pl.loop(0, BPW)
        def _(r):
            for d in range(D // LANES):
                sl = pl.ds(d * LANES, LANES)
                u = xu[r, sl]
                xi[r, sl] = u * xi[r, sl]
                xn[r, sl] = u * xn[r, sl]

        pltpu.sync_copy(xi, zp_h.at[pl.ds(base, BPW)])
        pltpu.sync_copy(xn, zn_h.at[pl.ds(base, BPW)])

    return dec(x, users, items, neg_items)


def _tc_loss(zp, zn, Wd1, bd1, wd2, bd2, kl):
    def body(zp_ref, zn_ref, w1_ref, b1_ref, w2_ref, b2_ref, kl_ref, o_ref):
        w1 = w1_ref[...]
        b1 = b1_ref[...]
        w2 = w2_ref[...]
        hp = jnp.maximum(jnp.dot(zp_ref[...], w1, preferred_element_type=jnp.float32) + b1, 0.0)
        hn = jnp.maximum(jnp.dot(zn_ref[...], w1, preferred_element_type=jnp.float32) + b1, 0.0)
        lp = jnp.sum(hp * w2, axis=1, keepdims=True) + b2_ref[0, 0]
        ln = jnp.sum(hn * w2, axis=1, keepdims=True) + b2_ref[0, 0]
        lr = jnp.mean(_softplus(-lp)) + jnp.mean(_softplus(ln))
        o_ref[...] = jnp.full((1, 1), 1.0, jnp.float32) * (lr + 0.1 * kl_ref[0, 0])

    return pl.pallas_call(
        body,
        out_shape=jax.ShapeDtypeStruct((1, 1), jnp.float32),
    )(zp, zn, Wd1, bd1, wd2, bd2, kl)


def kernel(edge_values, emb, W_mean, b_mean, W_std, b_std, Wd1, bd1, Wd2, bd2,
           noise, edge_index, users, items, neg_items):
    pad = N_CHUNKS * CHUNK - N_EDGES
    idx_pad = jnp.concatenate(
        [edge_index, jnp.zeros((2, pad), jnp.int32)], axis=1)
    ev_pad = jnp.concatenate(
        [jax.lax.bitcast_convert_type(edge_values, jnp.int32),
         jnp.zeros((pad,), jnp.int32)])
    packed = jnp.stack(
        [idx_pad[0].reshape(N_CHUNKS, CHUNK),
         idx_pad[1].reshape(N_CHUNKS, CHUNK),
         ev_pad.reshape(N_CHUNKS, CHUNK)], axis=1)
    p = _sc_spmm(packed, _pack_halves(emb))
    h1 = _tc_add(p)
    q = _sc_spmm(packed, h1)
    x, kl = _tc_encode(q, noise, W_mean, b_mean.reshape(1, D),
                       W_std, b_std.reshape(1, D))
    zp, zn = _sc_decode(x, users, items, neg_items)
    loss = _tc_loss(zp, zn, Wd1, bd1.reshape(1, D), Wd2.reshape(1, D),
                    bd2.reshape(1, 1), kl)
    return loss[0, 0]
